# Initial kernel scaffold; baseline (speedup 1.0000x reference)
import jax
import jax.numpy as jnp
from jax.experimental import pallas as pl


def kernel(x, edge_index, W1, b1, W2, b2):
    raise NotImplementedError("write your pallas kernel here")


# speedup vs baseline: 41.1291x; 41.1291x over previous
import jax
import jax.numpy as jnp
from jax import lax
from jax.experimental import pallas as pl
from jax.experimental.pallas import tpu as pltpu
from jax.experimental.pallas import tpu_sc as plsc

N = 10000
DF = 128
H = 16
C = 7
E = 320000

NC = 2
NS = 16
CHUNK = 128
CPW = 80
EPW = CHUNK * CPW
E_PAD = NC * NS * EPW
N_PAD = 10240
RPS = N_PAD // NS

ROWS_TC = 1000
GRID_TC = N // ROWS_TC

_mesh = plsc.VectorSubcoreMesh(core_axis_name="c", subcore_axis_name="s")
_sc_params = pltpu.CompilerParams(use_tc_tiling_on_sc=False)


def _deg_body(dst2_hbm, zeros_hbm, ones_hbm, out_hbm, dst_v, ones_v, acc_sh,
              sem):
    cid = lax.axis_index("c")
    sid = lax.axis_index("s")
    w = sid * NC + cid
    pltpu.sync_copy(zeros_hbm.at[pl.ds(sid * RPS, RPS)],
                    acc_sh.at[pl.ds(sid * RPS, RPS)])
    pltpu.sync_copy(ones_hbm, ones_v)
    pltpu.sync_copy(dst2_hbm.at[pl.ds(w * CPW, CPW)], dst_v)
    plsc.subcore_barrier()

    @pl.loop(0, CPW)
    def _(j):
        pltpu.sync_copy(ones_v, acc_sh.at[dst_v.at[j]], add=True)

    plsc.subcore_barrier()
    pltpu.sync_copy(acc_sh.at[pl.ds(sid * RPS, RPS)],
                    out_hbm.at[cid, pl.ds(sid * RPS, RPS)])


def _agg_body(g_hbm, src_hbm, dst2_hbm, zeros_hbm, out_hbm,
              src_v, dst_v, r0, r1, acc_sh, sg0, sg1):
    cid = lax.axis_index("c")
    sid = lax.axis_index("s")
    w = sid * NC + cid
    pltpu.sync_copy(zeros_hbm.at[pl.ds(sid * RPS, RPS)],
                    acc_sh.at[pl.ds(sid * RPS, RPS)])
    pltpu.sync_copy(src_hbm.at[pl.ds(w * EPW, EPW)], src_v)
    pltpu.sync_copy(dst2_hbm.at[pl.ds(w * CPW, CPW)], dst_v)
    plsc.subcore_barrier()

    pltpu.async_copy(g_hbm.at[src_v.at[pl.ds(0, CHUNK)]], r0, sg0)

    @pl.loop(0, CPW, step=2)
    def _(j):
        pltpu.async_copy(g_hbm.at[src_v.at[pl.ds((j + 1) * CHUNK, CHUNK)]],
                         r1, sg1)
        pltpu.make_async_copy(g_hbm.at[src_v.at[pl.ds(j * CHUNK, CHUNK)]],
                              r0, sg0).wait()
        pltpu.sync_copy(r0, acc_sh.at[dst_v.at[j]], add=True)

        @pl.when(j + 2 < CPW)
        def _():
            pltpu.async_copy(
                g_hbm.at[src_v.at[pl.ds((j + 2) * CHUNK, CHUNK)]], r0, sg0)

        pltpu.make_async_copy(
            g_hbm.at[src_v.at[pl.ds((j + 1) * CHUNK, CHUNK)]], r1, sg1).wait()
        pltpu.sync_copy(r1, acc_sh.at[dst_v.at[j + 1]], add=True)

    plsc.subcore_barrier()
    pltpu.sync_copy(acc_sh.at[pl.ds(sid * RPS, RPS)],
                    out_hbm.at[cid, pl.ds(sid * RPS, RPS)])


def _sc_deg(dst2, zeros, ones):
    return pl.kernel(
        _deg_body,
        out_type=jax.ShapeDtypeStruct((NC, N_PAD, H), jnp.float32),
        mesh=_mesh,
        scratch_types=[
            pltpu.VMEM((CPW, CHUNK), jnp.int32),
            pltpu.VMEM((CHUNK, H), jnp.float32),
            pltpu.VMEM_SHARED((N_PAD, H), jnp.float32),
            pltpu.SemaphoreType.DMA,
        ],
        compiler_params=_sc_params,
    )(dst2, zeros, ones)


def _sc_agg(g, src, dst2, zeros):
    return pl.kernel(
        _agg_body,
        out_type=jax.ShapeDtypeStruct((NC, N_PAD, H), jnp.float32),
        mesh=_mesh,
        scratch_types=[
            pltpu.VMEM((EPW,), jnp.int32),
            pltpu.VMEM((CPW, CHUNK), jnp.int32),
            pltpu.VMEM((CHUNK, H), jnp.float32),
            pltpu.VMEM((CHUNK, H), jnp.float32),
            pltpu.VMEM_SHARED((N_PAD, H), jnp.float32),
            pltpu.SemaphoreType.DMA,
            pltpu.SemaphoreType.DMA,
        ],
        compiler_params=_sc_params,
    )(g, src, dst2, zeros)


def _mm_body(x_ref, w_ref, o_ref):
    o_ref[...] = jnp.dot(x_ref[...], w_ref[...],
                         preferred_element_type=jnp.float32)


def _tc_mm(x, W1):
    return pl.pallas_call(
        _mm_body,
        grid=(GRID_TC,),
        in_specs=[pl.BlockSpec((ROWS_TC, DF), lambda i: (i, 0)),
                  pl.BlockSpec((DF, H), lambda i: (0, 0))],
        out_specs=pl.BlockSpec((ROWS_TC, H), lambda i: (i, 0)),
        out_shape=jax.ShapeDtypeStruct((N, H), jnp.float32),
    )(x, W1)


def _s1_body(deg_ref, h_ref, g_ref, dis_ref):
    deg = deg_ref[0] + deg_ref[1] + 1.0
    dis = lax.rsqrt(deg)
    dis_ref[...] = dis
    g_ref[...] = dis * h_ref[...]


def _tc_scale(deg2, h1):
    return pl.pallas_call(
        _s1_body,
        grid=(GRID_TC,),
        in_specs=[pl.BlockSpec((NC, ROWS_TC, H), lambda i: (0, i, 0)),
                  pl.BlockSpec((ROWS_TC, H), lambda i: (i, 0))],
        out_specs=[pl.BlockSpec((ROWS_TC, H), lambda i: (i, 0)),
                   pl.BlockSpec((ROWS_TC, H), lambda i: (i, 0))],
        out_shape=[jax.ShapeDtypeStruct((N, H), jnp.float32),
                   jax.ShapeDtypeStruct((N, H), jnp.float32)],
    )(deg2, h1)


def _s2_body(a_ref, g1_ref, dis_ref, w2_ref, b1_ref, g2_ref):
    dis = dis_ref[...]
    pre = dis * (a_ref[0] + a_ref[1] + g1_ref[...]) + b1_ref[...]
    h = jnp.maximum(pre, 0.0)
    g2_ref[...] = dis * jnp.dot(h, w2_ref[...],
                                preferred_element_type=jnp.float32)


def _tc_layer2(agg1, g1, dis, W2p, b1r):
    return pl.pallas_call(
        _s2_body,
        grid=(GRID_TC,),
        in_specs=[pl.BlockSpec((NC, ROWS_TC, H), lambda i: (0, i, 0)),
                  pl.BlockSpec((ROWS_TC, H), lambda i: (i, 0)),
                  pl.BlockSpec((ROWS_TC, H), lambda i: (i, 0)),
                  pl.BlockSpec((H, H), lambda i: (0, 0)),
                  pl.BlockSpec((1, H), lambda i: (0, 0))],
        out_specs=pl.BlockSpec((ROWS_TC, H), lambda i: (i, 0)),
        out_shape=jax.ShapeDtypeStruct((N, H), jnp.float32),
    )(agg1, g1, dis, W2p, b1r)


def _s3_body(a_ref, g2_ref, dis_ref, b2_ref, o_ref):
    logits = dis_ref[...] * (a_ref[0] + a_ref[1] + g2_ref[...]) + b2_ref[...]
    col = lax.broadcasted_iota(jnp.int32, logits.shape, 1)
    masked = jnp.where(col < C, logits, -jnp.inf)
    m = jnp.max(masked, axis=1, keepdims=True)
    s = jnp.sum(jnp.exp(masked - m), axis=1, keepdims=True)
    o_ref[...] = logits - (m + jnp.log(s))


def _tc_logsoftmax(agg2, g2, dis, b2r):
    return pl.pallas_call(
        _s3_body,
        grid=(GRID_TC,),
        in_specs=[pl.BlockSpec((NC, ROWS_TC, H), lambda i: (0, i, 0)),
                  pl.BlockSpec((ROWS_TC, H), lambda i: (i, 0)),
                  pl.BlockSpec((ROWS_TC, H), lambda i: (i, 0)),
                  pl.BlockSpec((1, H), lambda i: (0, 0))],
        out_specs=pl.BlockSpec((ROWS_TC, H), lambda i: (i, 0)),
        out_shape=jax.ShapeDtypeStruct((N, H), jnp.float32),
    )(agg2, g2, dis, b2r)


def kernel(x, edge_index, W1, b1, W2, b2):
    src = edge_index[0].astype(jnp.int32)
    dst = edge_index[1].astype(jnp.int32)
    npad = E_PAD - E
    pad_src = jnp.zeros((npad,), jnp.int32)
    pad_dst = N + (jnp.arange(npad, dtype=jnp.int32) % (N_PAD - N))
    srcp = jnp.concatenate([src, pad_src])
    dst2 = jnp.concatenate([dst, pad_dst]).reshape(E_PAD // CHUNK, CHUNK)

    zeros = jnp.zeros((N_PAD, H), jnp.float32)
    ones = jnp.ones((CHUNK, H), jnp.float32)
    W2p = jnp.zeros((H, H), jnp.float32).at[:, :C].set(W2)
    b1r = b1.reshape(1, H)
    b2r = jnp.zeros((1, H), jnp.float32).at[0, :C].set(b2)

    deg2 = _sc_deg(dst2, zeros, ones)
    h1 = _tc_mm(x, W1)
    g1, dis = _tc_scale(deg2, h1)
    agg1 = _sc_agg(g1, srcp, dst2, zeros)
    g2 = _tc_layer2(agg1, g1, dis, W2p, b1r)
    agg2 = _sc_agg(g2, srcp, dst2, zeros)
    out16 = _tc_logsoftmax(agg2, g2, dis, b2r)
    return out16[:, :C]


# speedup vs baseline: 43.0229x; 1.0460x over previous
import jax
import jax.numpy as jnp
from jax import lax
from jax.experimental import pallas as pl
from jax.experimental.pallas import tpu as pltpu
from jax.experimental.pallas import tpu_sc as plsc

N = 10000
DF = 128
H = 16
C = 7
E = 320000

NC = 2
NS = 16
CB = 8
CHUNK = CB * 128
CPW = 10
EPW = CHUNK * CPW
E_PAD = NC * NS * EPW
N_PAD = 10240
RPS = N_PAD // NS

ROWS_TC = 1000
GRID_TC = N // ROWS_TC

_mesh = plsc.VectorSubcoreMesh(core_axis_name="c", subcore_axis_name="s")
_sc_params = pltpu.CompilerParams(use_tc_tiling_on_sc=False)


def _deg_body(dst_hbm, zeros_hbm, ones_hbm, out_hbm, dst_v, ones_v, acc_sh,
              sem):
    cid = lax.axis_index("c")
    sid = lax.axis_index("s")
    w = sid * NC + cid
    pltpu.sync_copy(zeros_hbm.at[pl.ds(sid * RPS, RPS)],
                    acc_sh.at[pl.ds(sid * RPS, RPS)])
    pltpu.sync_copy(ones_hbm, ones_v)
    pltpu.sync_copy(dst_hbm.at[pl.ds(w * EPW, EPW)], dst_v)
    plsc.subcore_barrier()

    @pl.loop(0, CPW)
    def _(j):
        pltpu.async_copy(ones_v, acc_sh.at[dst_v.at[pl.ds(j * CHUNK, CHUNK)]],
                         sem, add=True)

    @pl.loop(0, CPW)
    def _(j):
        pltpu.make_async_copy(
            ones_v, acc_sh.at[dst_v.at[pl.ds(j * CHUNK, CHUNK)]], sem).wait()

    plsc.subcore_barrier()
    pltpu.sync_copy(acc_sh.at[pl.ds(sid * RPS, RPS)],
                    out_hbm.at[cid, pl.ds(sid * RPS, RPS)])


def _agg_body(g_hbm, src_hbm, dst_hbm, zeros_hbm, out_hbm,
              src_v, dst_v, r0, r1, acc_sh, sg0, sg1):
    cid = lax.axis_index("c")
    sid = lax.axis_index("s")
    w = sid * NC + cid
    pltpu.sync_copy(zeros_hbm.at[pl.ds(sid * RPS, RPS)],
                    acc_sh.at[pl.ds(sid * RPS, RPS)])
    pltpu.sync_copy(src_hbm.at[pl.ds(w * EPW, EPW)], src_v)
    pltpu.sync_copy(dst_hbm.at[pl.ds(w * EPW, EPW)], dst_v)
    plsc.subcore_barrier()

    pltpu.async_copy(g_hbm.at[src_v.at[pl.ds(0, CHUNK)]], r0, sg0)

    @pl.loop(0, CPW, step=2)
    def _(j):
        pltpu.async_copy(g_hbm.at[src_v.at[pl.ds((j + 1) * CHUNK, CHUNK)]],
                         r1, sg1)
        pltpu.make_async_copy(g_hbm.at[src_v.at[pl.ds(j * CHUNK, CHUNK)]],
                              r0, sg0).wait()
        pltpu.sync_copy(r0, acc_sh.at[dst_v.at[pl.ds(j * CHUNK, CHUNK)]],
                        add=True)

        @pl.when(j + 2 < CPW)
        def _():
            pltpu.async_copy(
                g_hbm.at[src_v.at[pl.ds((j + 2) * CHUNK, CHUNK)]], r0, sg0)

        pltpu.make_async_copy(
            g_hbm.at[src_v.at[pl.ds((j + 1) * CHUNK, CHUNK)]], r1, sg1).wait()
        pltpu.sync_copy(r1, acc_sh.at[dst_v.at[pl.ds((j + 1) * CHUNK, CHUNK)]],
                        add=True)

    plsc.subcore_barrier()
    pltpu.sync_copy(acc_sh.at[pl.ds(sid * RPS, RPS)],
                    out_hbm.at[cid, pl.ds(sid * RPS, RPS)])


def _sc_deg(dstp, zeros, ones):
    return pl.kernel(
        _deg_body,
        out_type=jax.ShapeDtypeStruct((NC, N_PAD, H), jnp.float32),
        mesh=_mesh,
        scratch_types=[
            pltpu.VMEM((EPW,), jnp.int32),
            pltpu.VMEM((CHUNK, H), jnp.float32),
            pltpu.VMEM_SHARED((N_PAD, H), jnp.float32),
            pltpu.SemaphoreType.DMA,
        ],
        compiler_params=_sc_params,
    )(dstp, zeros, ones)


def _sc_agg(g, srcp, dstp, zeros):
    return pl.kernel(
        _agg_body,
        out_type=jax.ShapeDtypeStruct((NC, N_PAD, H), jnp.float32),
        mesh=_mesh,
        scratch_types=[
            pltpu.VMEM((EPW,), jnp.int32),
            pltpu.VMEM((EPW,), jnp.int32),
            pltpu.VMEM((CHUNK, H), jnp.float32),
            pltpu.VMEM((CHUNK, H), jnp.float32),
            pltpu.VMEM_SHARED((N_PAD, H), jnp.float32),
            pltpu.SemaphoreType.DMA,
            pltpu.SemaphoreType.DMA,
        ],
        compiler_params=_sc_params,
    )(g, srcp, dstp, zeros)


def _s1_body(deg_ref, x_ref, w_ref, g_ref, dis_ref):
    h = jnp.dot(x_ref[...], w_ref[...], preferred_element_type=jnp.float32)
    deg = deg_ref[0] + deg_ref[1] + 1.0
    dis = lax.rsqrt(deg)
    dis_ref[...] = dis
    g_ref[...] = dis * h


def _tc_mm_scale(deg2, x, W1):
    return pl.pallas_call(
        _s1_body,
        grid=(GRID_TC,),
        in_specs=[pl.BlockSpec((NC, ROWS_TC, H), lambda i: (0, i, 0)),
                  pl.BlockSpec((ROWS_TC, DF), lambda i: (i, 0)),
                  pl.BlockSpec((DF, H), lambda i: (0, 0))],
        out_specs=[pl.BlockSpec((ROWS_TC, H), lambda i: (i, 0)),
                   pl.BlockSpec((ROWS_TC, H), lambda i: (i, 0))],
        out_shape=[jax.ShapeDtypeStruct((N, H), jnp.float32),
                   jax.ShapeDtypeStruct((N, H), jnp.float32)],
    )(deg2, x, W1)


def _s2_body(a_ref, g1_ref, dis_ref, w2_ref, b1_ref, g2_ref):
    dis = dis_ref[...]
    pre = dis * (a_ref[0] + a_ref[1] + g1_ref[...]) + b1_ref[...]
    h = jnp.maximum(pre, 0.0)
    g2_ref[...] = dis * jnp.dot(h, w2_ref[...],
                                preferred_element_type=jnp.float32)


def _tc_layer2(agg1, g1, dis, W2p, b1r):
    return pl.pallas_call(
        _s2_body,
        grid=(GRID_TC,),
        in_specs=[pl.BlockSpec((NC, ROWS_TC, H), lambda i: (0, i, 0)),
                  pl.BlockSpec((ROWS_TC, H), lambda i: (i, 0)),
                  pl.BlockSpec((ROWS_TC, H), lambda i: (i, 0)),
                  pl.BlockSpec((H, H), lambda i: (0, 0)),
                  pl.BlockSpec((1, H), lambda i: (0, 0))],
        out_specs=pl.BlockSpec((ROWS_TC, H), lambda i: (i, 0)),
        out_shape=jax.ShapeDtypeStruct((N, H), jnp.float32),
    )(agg1, g1, dis, W2p, b1r)


def _s3_body(a_ref, g2_ref, dis_ref, b2_ref, o_ref):
    logits = dis_ref[...] * (a_ref[0] + a_ref[1] + g2_ref[...]) + b2_ref[...]
    col = lax.broadcasted_iota(jnp.int32, logits.shape, 1)
    masked = jnp.where(col < C, logits, -jnp.inf)
    m = jnp.max(masked, axis=1, keepdims=True)
    s = jnp.sum(jnp.exp(masked - m), axis=1, keepdims=True)
    o_ref[...] = logits - (m + jnp.log(s))


def _tc_logsoftmax(agg2, g2, dis, b2r):
    return pl.pallas_call(
        _s3_body,
        grid=(GRID_TC,),
        in_specs=[pl.BlockSpec((NC, ROWS_TC, H), lambda i: (0, i, 0)),
                  pl.BlockSpec((ROWS_TC, H), lambda i: (i, 0)),
                  pl.BlockSpec((ROWS_TC, H), lambda i: (i, 0)),
                  pl.BlockSpec((1, H), lambda i: (0, 0))],
        out_specs=pl.BlockSpec((ROWS_TC, H), lambda i: (i, 0)),
        out_shape=jax.ShapeDtypeStruct((N, H), jnp.float32),
    )(agg2, g2, dis, b2r)


def kernel(x, edge_index, W1, b1, W2, b2):
    src = edge_index[0].astype(jnp.int32)
    dst = edge_index[1].astype(jnp.int32)
    npad = E_PAD - E
    pad_src = jnp.zeros((npad,), jnp.int32)
    pad_dst = N + (jnp.arange(npad, dtype=jnp.int32) % (N_PAD - N))
    srcp = jnp.concatenate([src, pad_src])
    dstp = jnp.concatenate([dst, pad_dst])

    zeros = jnp.zeros((N_PAD, H), jnp.float32)
    ones = jnp.ones((CHUNK, H), jnp.float32)
    W2p = jnp.zeros((H, H), jnp.float32).at[:, :C].set(W2)
    b1r = b1.reshape(1, H)
    b2r = jnp.zeros((1, H), jnp.float32).at[0, :C].set(b2)

    deg2 = _sc_deg(dstp, zeros, ones)
    g1, dis = _tc_mm_scale(deg2, x, W1)
    agg1 = _sc_agg(g1, srcp, dstp, zeros)
    g2 = _tc_layer2(agg1, g1, dis, W2p, b1r)
    agg2 = _sc_agg(g2, srcp, dstp, zeros)
    out16 = _tc_logsoftmax(agg2, g2, dis, b2r)
    return out16[:, :C]


# speedup vs baseline: 57.8575x; 1.3448x over previous
import jax
import jax.numpy as jnp
from jax import lax
from jax.experimental import pallas as pl
from jax.experimental.pallas import tpu as pltpu
from jax.experimental.pallas import tpu_sc as plsc

N = 10000
DF = 128
H = 16
C = 7
E = 320000

NC = 2
NS = 16
CB = 8
CHUNK = CB * 128
CPW = 10
EPW = CHUNK * CPW
E_PAD = NC * NS * EPW
N_PAD = 10240
RPS = N_PAD // NS

ROWS_TC = 1000
GRID_TC = N // ROWS_TC

_mesh = plsc.VectorSubcoreMesh(core_axis_name="c", subcore_axis_name="s")
_sc_params = pltpu.CompilerParams(use_tc_tiling_on_sc=False)


def _deg_body(dst_hbm, zeros_hbm, ones_hbm, out_hbm, dst_v, ones_v, acc_sh,
              sem):
    cid = lax.axis_index("c")
    sid = lax.axis_index("s")
    w = sid * NC + cid
    pltpu.sync_copy(zeros_hbm.at[pl.ds(sid * RPS, RPS)],
                    acc_sh.at[pl.ds(sid * RPS, RPS)])
    pltpu.sync_copy(ones_hbm, ones_v)
    pltpu.sync_copy(dst_hbm.at[pl.ds(w * EPW, EPW)], dst_v)
    plsc.subcore_barrier()

    @pl.loop(0, CPW)
    def _(j):
        pltpu.async_copy(ones_v, acc_sh.at[dst_v.at[pl.ds(j * CHUNK, CHUNK)]],
                         sem, add=True)

    @pl.loop(0, CPW)
    def _(j):
        pltpu.make_async_copy(
            ones_v, acc_sh.at[dst_v.at[pl.ds(j * CHUNK, CHUNK)]], sem).wait()

    plsc.subcore_barrier()
    pltpu.sync_copy(acc_sh.at[pl.ds(sid * RPS, RPS)],
                    out_hbm.at[cid, pl.ds(sid * RPS, RPS)])


def _agg_body(g_hbm, src_hbm, dst_hbm, zeros_hbm, out_hbm,
              src_v, dst_v, r0, r1, g_sh, acc_sh, sg0, sg1):
    cid = lax.axis_index("c")
    sid = lax.axis_index("s")
    w = sid * NC + cid
    gps = N // NS
    pltpu.async_copy(g_hbm.at[pl.ds(sid * gps, gps)],
                     g_sh.at[pl.ds(sid * gps, gps)], sg1)
    pltpu.sync_copy(zeros_hbm.at[pl.ds(sid * RPS, RPS)],
                    acc_sh.at[pl.ds(sid * RPS, RPS)])
    pltpu.sync_copy(src_hbm.at[pl.ds(w * EPW, EPW)], src_v)
    pltpu.sync_copy(dst_hbm.at[pl.ds(w * EPW, EPW)], dst_v)
    pltpu.make_async_copy(g_hbm.at[pl.ds(sid * gps, gps)],
                          g_sh.at[pl.ds(sid * gps, gps)], sg1).wait()
    plsc.subcore_barrier()

    pltpu.async_copy(g_sh.at[src_v.at[pl.ds(0, CHUNK)]], r0, sg0)

    @pl.loop(0, CPW, step=2)
    def _(j):
        pltpu.async_copy(g_sh.at[src_v.at[pl.ds((j + 1) * CHUNK, CHUNK)]],
                         r1, sg1)
        pltpu.make_async_copy(g_sh.at[src_v.at[pl.ds(j * CHUNK, CHUNK)]],
                              r0, sg0).wait()
        pltpu.sync_copy(r0, acc_sh.at[dst_v.at[pl.ds(j * CHUNK, CHUNK)]],
                        add=True)

        @pl.when(j + 2 < CPW)
        def _():
            pltpu.async_copy(
                g_sh.at[src_v.at[pl.ds((j + 2) * CHUNK, CHUNK)]], r0, sg0)

        pltpu.make_async_copy(
            g_sh.at[src_v.at[pl.ds((j + 1) * CHUNK, CHUNK)]], r1, sg1).wait()
        pltpu.sync_copy(r1, acc_sh.at[dst_v.at[pl.ds((j + 1) * CHUNK, CHUNK)]],
                        add=True)

    plsc.subcore_barrier()
    pltpu.sync_copy(acc_sh.at[pl.ds(sid * RPS, RPS)],
                    out_hbm.at[cid, pl.ds(sid * RPS, RPS)])


def _sc_deg(dstp, zeros, ones):
    return pl.kernel(
        _deg_body,
        out_type=jax.ShapeDtypeStruct((NC, N_PAD, H), jnp.float32),
        mesh=_mesh,
        scratch_types=[
            pltpu.VMEM((EPW,), jnp.int32),
            pltpu.VMEM((CHUNK, H), jnp.float32),
            pltpu.VMEM_SHARED((N_PAD, H), jnp.float32),
            pltpu.SemaphoreType.DMA,
        ],
        compiler_params=_sc_params,
    )(dstp, zeros, ones)


def _sc_agg(g, srcp, dstp, zeros):
    return pl.kernel(
        _agg_body,
        out_type=jax.ShapeDtypeStruct((NC, N_PAD, H), jnp.float32),
        mesh=_mesh,
        scratch_types=[
            pltpu.VMEM((EPW,), jnp.int32),
            pltpu.VMEM((EPW,), jnp.int32),
            pltpu.VMEM((CHUNK, H), jnp.float32),
            pltpu.VMEM((CHUNK, H), jnp.float32),
            pltpu.VMEM_SHARED((N, H), jnp.float32),
            pltpu.VMEM_SHARED((N_PAD, H), jnp.float32),
            pltpu.SemaphoreType.DMA,
            pltpu.SemaphoreType.DMA,
        ],
        compiler_params=_sc_params,
    )(g, srcp, dstp, zeros)


def _s1_body(deg_ref, x_ref, w_ref, g_ref, dis_ref):
    h = jnp.dot(x_ref[...], w_ref[...], preferred_element_type=jnp.float32)
    deg = deg_ref[0] + deg_ref[1] + 1.0
    dis = lax.rsqrt(deg)
    dis_ref[...] = dis
    g_ref[...] = dis * h


def _tc_mm_scale(deg2, x, W1):
    return pl.pallas_call(
        _s1_body,
        grid=(GRID_TC,),
        in_specs=[pl.BlockSpec((NC, ROWS_TC, H), lambda i: (0, i, 0)),
                  pl.BlockSpec((ROWS_TC, DF), lambda i: (i, 0)),
                  pl.BlockSpec((DF, H), lambda i: (0, 0))],
        out_specs=[pl.BlockSpec((ROWS_TC, H), lambda i: (i, 0)),
                   pl.BlockSpec((ROWS_TC, H), lambda i: (i, 0))],
        out_shape=[jax.ShapeDtypeStruct((N, H), jnp.float32),
                   jax.ShapeDtypeStruct((N, H), jnp.float32)],
    )(deg2, x, W1)


def _s2_body(a_ref, g1_ref, dis_ref, w2_ref, b1_ref, g2_ref):
    dis = dis_ref[...]
    pre = dis * (a_ref[0] + a_ref[1] + g1_ref[...]) + b1_ref[...]
    h = jnp.maximum(pre, 0.0)
    g2_ref[...] = dis * jnp.dot(h, w2_ref[...],
                                preferred_element_type=jnp.float32)


def _tc_layer2(agg1, g1, dis, W2p, b1r):
    return pl.pallas_call(
        _s2_body,
        grid=(GRID_TC,),
        in_specs=[pl.BlockSpec((NC, ROWS_TC, H), lambda i: (0, i, 0)),
                  pl.BlockSpec((ROWS_TC, H), lambda i: (i, 0)),
                  pl.BlockSpec((ROWS_TC, H), lambda i: (i, 0)),
                  pl.BlockSpec((H, H), lambda i: (0, 0)),
                  pl.BlockSpec((1, H), lambda i: (0, 0))],
        out_specs=pl.BlockSpec((ROWS_TC, H), lambda i: (i, 0)),
        out_shape=jax.ShapeDtypeStruct((N, H), jnp.float32),
    )(agg1, g1, dis, W2p, b1r)


def _s3_body(a_ref, g2_ref, dis_ref, b2_ref, o_ref):
    logits = dis_ref[...] * (a_ref[0] + a_ref[1] + g2_ref[...]) + b2_ref[...]
    col = lax.broadcasted_iota(jnp.int32, logits.shape, 1)
    masked = jnp.where(col < C, logits, -jnp.inf)
    m = jnp.max(masked, axis=1, keepdims=True)
    s = jnp.sum(jnp.exp(masked - m), axis=1, keepdims=True)
    o_ref[...] = logits - (m + jnp.log(s))


def _tc_logsoftmax(agg2, g2, dis, b2r):
    return pl.pallas_call(
        _s3_body,
        grid=(GRID_TC,),
        in_specs=[pl.BlockSpec((NC, ROWS_TC, H), lambda i: (0, i, 0)),
                  pl.BlockSpec((ROWS_TC, H), lambda i: (i, 0)),
                  pl.BlockSpec((ROWS_TC, H), lambda i: (i, 0)),
                  pl.BlockSpec((1, H), lambda i: (0, 0))],
        out_specs=pl.BlockSpec((ROWS_TC, H), lambda i: (i, 0)),
        out_shape=jax.ShapeDtypeStruct((N, H), jnp.float32),
    )(agg2, g2, dis, b2r)


def kernel(x, edge_index, W1, b1, W2, b2):
    src = edge_index[0].astype(jnp.int32)
    dst = edge_index[1].astype(jnp.int32)
    npad = E_PAD - E
    pad_src = jnp.zeros((npad,), jnp.int32)
    pad_dst = N + (jnp.arange(npad, dtype=jnp.int32) % (N_PAD - N))
    srcp = jnp.concatenate([src, pad_src])
    dstp = jnp.concatenate([dst, pad_dst])

    zeros = jnp.zeros((N_PAD, H), jnp.float32)
    ones = jnp.ones((CHUNK, H), jnp.float32)
    W2p = jnp.zeros((H, H), jnp.float32).at[:, :C].set(W2)
    b1r = b1.reshape(1, H)
    b2r = jnp.zeros((1, H), jnp.float32).at[0, :C].set(b2)

    deg2 = _sc_deg(dstp, zeros, ones)
    g1, dis = _tc_mm_scale(deg2, x, W1)
    agg1 = _sc_agg(g1, srcp, dstp, zeros)
    g2 = _tc_layer2(agg1, g1, dis, W2p, b1r)
    agg2 = _sc_agg(g2, srcp, dstp, zeros)
    out16 = _tc_logsoftmax(agg2, g2, dis, b2r)
    return out16[:, :C]


# speedup vs baseline: 77.6328x; 1.3418x over previous
import jax
import jax.numpy as jnp
from jax import lax
from jax.experimental import pallas as pl
from jax.experimental.pallas import tpu as pltpu
from jax.experimental.pallas import tpu_sc as plsc

N = 10000
DF = 128
H = 16
C = 7
E = 320000

NC = 2
NS = 16
CHUNK = 1024
CPW = 10
EPW = CHUNK * CPW
E_PAD = NC * NS * EPW
N_PAD = 10240
RPS = N_PAD // NS

ROWS = 1024
ROWS8 = ROWS // 8
GRID_TC = N_PAD // ROWS
NP8 = N_PAD * H // 128

_mesh = plsc.VectorSubcoreMesh(core_axis_name="c", subcore_axis_name="s")
_sc_params = pltpu.CompilerParams(use_tc_tiling_on_sc=False)


def _deg_body(dst_hbm, ones_hbm, out_hbm, dst_v, ones_v, acc_sh, sem):
    cid = lax.axis_index("c")
    sid = lax.axis_index("s")
    w = sid * NC + cid
    pltpu.sync_copy(ones_hbm.at[pl.ds(sid * RPS, RPS)],
                    acc_sh.at[pl.ds(sid * RPS, RPS)])
    pltpu.sync_copy(ones_hbm.at[pl.ds(0, CHUNK)], ones_v)
    pltpu.sync_copy(dst_hbm.at[pl.ds(w * EPW, EPW)], dst_v)
    plsc.subcore_barrier()

    @pl.loop(0, CPW)
    def _(j):
        pltpu.async_copy(ones_v, acc_sh.at[dst_v.at[pl.ds(j * CHUNK, CHUNK)]],
                         sem, add=True)

    @pl.loop(0, CPW)
    def _(j):
        pltpu.make_async_copy(
            ones_v, acc_sh.at[dst_v.at[pl.ds(j * CHUNK, CHUNK)]], sem).wait()

    plsc.subcore_barrier()
    pltpu.sync_copy(acc_sh.at[pl.ds(sid * RPS, RPS)],
                    out_hbm.at[cid, pl.ds(sid * RPS, RPS)])


def _agg_body(g_hbm, src_hbm, dst_hbm, out_hbm,
              src_v, dst_v, r0, r1, g_sh, acc_sh, sg0, sg1):
    cid = lax.axis_index("c")
    sid = lax.axis_index("s")
    w = sid * NC + cid
    pltpu.async_copy(g_hbm.at[pl.ds(sid * RPS, RPS)],
                     g_sh.at[pl.ds(sid * RPS, RPS)], sg1)
    pltpu.async_copy(g_hbm.at[pl.ds(sid * RPS, RPS)],
                     acc_sh.at[pl.ds(sid * RPS, RPS)], sg0)
    pltpu.sync_copy(src_hbm.at[pl.ds(w * EPW, EPW)], src_v)
    pltpu.sync_copy(dst_hbm.at[pl.ds(w * EPW, EPW)], dst_v)
    pltpu.make_async_copy(g_hbm.at[pl.ds(sid * RPS, RPS)],
                          g_sh.at[pl.ds(sid * RPS, RPS)], sg1).wait()
    pltpu.make_async_copy(g_hbm.at[pl.ds(sid * RPS, RPS)],
                          acc_sh.at[pl.ds(sid * RPS, RPS)], sg0).wait()
    plsc.subcore_barrier()

    pltpu.async_copy(g_sh.at[src_v.at[pl.ds(0, CHUNK)]], r0, sg0)

    @pl.loop(0, CPW, step=2)
    def _(j):
        pltpu.async_copy(g_sh.at[src_v.at[pl.ds((j + 1) * CHUNK, CHUNK)]],
                         r1, sg1)
        pltpu.make_async_copy(g_sh.at[src_v.at[pl.ds(j * CHUNK, CHUNK)]],
                              r0, sg0).wait()
        pltpu.sync_copy(r0, acc_sh.at[dst_v.at[pl.ds(j * CHUNK, CHUNK)]],
                        add=True)

        @pl.when(j + 2 < CPW)
        def _():
            pltpu.async_copy(
                g_sh.at[src_v.at[pl.ds((j + 2) * CHUNK, CHUNK)]], r0, sg0)

        pltpu.make_async_copy(
            g_sh.at[src_v.at[pl.ds((j + 1) * CHUNK, CHUNK)]], r1, sg1).wait()
        pltpu.sync_copy(r1, acc_sh.at[dst_v.at[pl.ds((j + 1) * CHUNK, CHUNK)]],
                        add=True)

    plsc.subcore_barrier()
    pltpu.sync_copy(acc_sh.at[pl.ds(sid * RPS, RPS)],
                    out_hbm.at[cid, pl.ds(sid * RPS, RPS)])


def _sc_deg(dstp, ones2):
    return pl.kernel(
        _deg_body,
        out_type=jax.ShapeDtypeStruct((NC, N_PAD, H), jnp.float32),
        mesh=_mesh,
        scratch_types=[
            pltpu.VMEM((EPW,), jnp.int32),
            pltpu.VMEM((CHUNK, H), jnp.float32),
            pltpu.VMEM_SHARED((N_PAD, H), jnp.float32),
            pltpu.SemaphoreType.DMA,
        ],
        compiler_params=_sc_params,
    )(dstp, ones2)


def _sc_agg(g, srcp, dstp):
    return pl.kernel(
        _agg_body,
        out_type=jax.ShapeDtypeStruct((NC, N_PAD, H), jnp.float32),
        mesh=_mesh,
        scratch_types=[
            pltpu.VMEM((EPW,), jnp.int32),
            pltpu.VMEM((EPW,), jnp.int32),
            pltpu.VMEM((CHUNK, H), jnp.float32),
            pltpu.VMEM((CHUNK, H), jnp.float32),
            pltpu.VMEM_SHARED((N_PAD, H), jnp.float32),
            pltpu.VMEM_SHARED((N_PAD, H), jnp.float32),
            pltpu.SemaphoreType.DMA,
            pltpu.SemaphoreType.DMA,
        ],
        compiler_params=_sc_params,
    )(g, srcp, dstp)


def _s1_body(deg_ref, xs_ref, w_ref, g_ref, dis_ref):
    h = jnp.dot(xs_ref[...], w_ref[...], preferred_element_type=jnp.float32)
    deg = deg_ref[0] + deg_ref[1] - 1.0
    dis = lax.rsqrt(deg)
    dis_ref[...] = dis
    g_ref[...] = dis * h


def _tc_mm_scale(deg2v, xs, W1s):
    return pl.pallas_call(
        _s1_body,
        grid=(GRID_TC,),
        in_specs=[pl.BlockSpec((NC, ROWS8, 128), lambda i: (0, i, 0)),
                  pl.BlockSpec((ROWS8, 8 * DF), lambda i: (i, 0)),
                  pl.BlockSpec((8 * DF, 128), lambda i: (0, 0))],
        out_specs=[pl.BlockSpec((ROWS8, 128), lambda i: (i, 0)),
                   pl.BlockSpec((ROWS8, 128), lambda i: (i, 0))],
        out_shape=[jax.ShapeDtypeStruct((NP8, 128), jnp.float32),
                   jax.ShapeDtypeStruct((NP8, 128), jnp.float32)],
    )(deg2v, xs, W1s)


def _s2_body(a_ref, g1_ref, dis_ref, w2_ref, b1_ref, g2_ref):
    dis = dis_ref[...]
    pre = dis * (a_ref[0] + a_ref[1] - g1_ref[...]) + b1_ref[...]
    h = jnp.maximum(pre, 0.0)
    h2 = jnp.dot(h, w2_ref[...], preferred_element_type=jnp.float32)
    g2_ref[...] = dis * h2


def _tc_layer2(agg1v, g1v, dis, W2bd, b1t):
    return pl.pallas_call(
        _s2_body,
        grid=(GRID_TC,),
        in_specs=[pl.BlockSpec((NC, ROWS8, 128), lambda i: (0, i, 0)),
                  pl.BlockSpec((ROWS8, 128), lambda i: (i, 0)),
                  pl.BlockSpec((ROWS8, 128), lambda i: (i, 0)),
                  pl.BlockSpec((128, 128), lambda i: (0, 0)),
                  pl.BlockSpec((1, 128), lambda i: (0, 0))],
        out_specs=pl.BlockSpec((ROWS8, 128), lambda i: (i, 0)),
        out_shape=jax.ShapeDtypeStruct((NP8, 128), jnp.float32),
    )(agg1v, g1v, dis, W2bd, b1t)


def _s3_body(a_ref, g2_ref, dis_ref, b2_ref, seg_ref, o_ref):
    logits = (dis_ref[...] * (a_ref[0] + a_ref[1] - g2_ref[...])
              + b2_ref[...])
    col = lax.broadcasted_iota(jnp.int32, logits.shape, 1)
    masked = jnp.where(col % H < C, logits, -jnp.inf)
    m = jnp.max(masked, axis=1, keepdims=True)
    e = jnp.exp(masked - m)
    s = jnp.dot(e, seg_ref[...], preferred_element_type=jnp.float32)
    o_ref[...] = logits - (m + jnp.log(s))


def _tc_logsoftmax(agg2v, g2v, dis, b2t, seg):
    return pl.pallas_call(
        _s3_body,
        grid=(GRID_TC,),
        in_specs=[pl.BlockSpec((NC, ROWS8, 128), lambda i: (0, i, 0)),
                  pl.BlockSpec((ROWS8, 128), lambda i: (i, 0)),
                  pl.BlockSpec((ROWS8, 128), lambda i: (i, 0)),
                  pl.BlockSpec((1, 128), lambda i: (0, 0)),
                  pl.BlockSpec((128, 128), lambda i: (0, 0))],
        out_specs=pl.BlockSpec((ROWS8, 128), lambda i: (i, 0)),
        out_shape=jax.ShapeDtypeStruct((NP8, 128), jnp.float32),
    )(agg2v, g2v, dis, b2t, seg)


def kernel(x, edge_index, W1, b1, W2, b2):
    src = edge_index[0].astype(jnp.int32)
    dst = edge_index[1].astype(jnp.int32)
    npad = E_PAD - E
    pad_src = jnp.zeros((npad,), jnp.int32)
    pad_dst = N + (jnp.arange(npad, dtype=jnp.int32) % (N_PAD - N))
    srcp = jnp.concatenate([src, pad_src])
    dstp = jnp.concatenate([dst, pad_dst])

    x16 = jnp.pad(x.astype(jnp.bfloat16), ((0, N_PAD - N), (0, 0)))
    xs = x16.reshape(NP8, 8 * DF)
    W116 = W1.astype(jnp.bfloat16)
    W1s = jnp.zeros((8 * DF, 128), jnp.bfloat16)
    W2p = jnp.zeros((H, H), jnp.float32).at[:, :C].set(W2)
    W2bd = jnp.zeros((128, 128), jnp.float32)
    for j in range(8):
        W1s = W1s.at[j * DF:(j + 1) * DF, j * H:(j + 1) * H].set(W116)
        W2bd = W2bd.at[j * H:(j + 1) * H, j * H:(j + 1) * H].set(W2p)
    ones2 = jnp.ones((N_PAD, H), jnp.float32)
    b1t = jnp.tile(b1, 8).reshape(1, 128)
    b2t = jnp.tile(jnp.zeros((H,), jnp.float32).at[:C].set(b2), 8)
    b2t = b2t.reshape(1, 128)
    lane = jnp.arange(128)
    seg = (lane[:, None] // H == lane[None, :] // H).astype(jnp.float32)

    deg2 = _sc_deg(dstp, ones2)
    deg2v = deg2.reshape(NC, NP8, 128)
    g1v, dis = _tc_mm_scale(deg2v, xs, W1s)
    agg1 = _sc_agg(g1v.reshape(N_PAD, H), srcp, dstp)
    g2v = _tc_layer2(agg1.reshape(NC, NP8, 128), g1v, dis, W2bd, b1t)
    agg2 = _sc_agg(g2v.reshape(N_PAD, H), srcp, dstp)
    out128 = _tc_logsoftmax(agg2.reshape(NC, NP8, 128), g2v, dis, b2t, seg)
    return out128.reshape(N_PAD, H)[:N, :C]


# speedup vs baseline: 90.7524x; 1.1690x over previous
import jax
import jax.numpy as jnp
from jax import lax
from jax.experimental import pallas as pl
from jax.experimental.pallas import tpu as pltpu
from jax.experimental.pallas import tpu_sc as plsc

N = 10000
DF = 128
H = 16
C = 7
E = 320000

NC = 2
NS = 16
CHUNK = 1000
CPW = 10
EPW = CHUNK * CPW
N_PAD = 10240
RPS = N_PAD // NS

ROWS = 1024
ROWS8 = ROWS // 8
GRID_TC = N_PAD // ROWS
NP8 = N_PAD * H // 128

_mesh = plsc.VectorSubcoreMesh(core_axis_name="c", subcore_axis_name="s")
_sc_params = pltpu.CompilerParams(use_tc_tiling_on_sc=False)


def _deg_body(ei_hbm, ones_hbm, out_hbm, dst_v, ones_v, acc_sh, sem):
    cid = lax.axis_index("c")
    sid = lax.axis_index("s")
    w = sid * NC + cid
    pltpu.sync_copy(ones_hbm.at[pl.ds(sid * RPS, RPS)],
                    acc_sh.at[pl.ds(sid * RPS, RPS)])
    pltpu.sync_copy(ones_hbm.at[pl.ds(0, CHUNK)], ones_v)
    pltpu.sync_copy(ei_hbm.at[1, pl.ds(w * EPW, EPW)], dst_v)
    plsc.subcore_barrier()

    @pl.loop(0, CPW)
    def _(j):
        pltpu.async_copy(ones_v, acc_sh.at[dst_v.at[pl.ds(j * CHUNK, CHUNK)]],
                         sem, add=True)

    @pl.loop(0, CPW)
    def _(j):
        pltpu.make_async_copy(
            ones_v, acc_sh.at[dst_v.at[pl.ds(j * CHUNK, CHUNK)]], sem).wait()

    plsc.subcore_barrier()
    pltpu.sync_copy(acc_sh.at[pl.ds(sid * RPS, RPS)],
                    out_hbm.at[cid, pl.ds(sid * RPS, RPS)])


def _agg_body(g_hbm, ei_hbm, out_hbm,
              src_v, dst_v, r0, r1, g_sh, acc_sh, sg0, sg1):
    cid = lax.axis_index("c")
    sid = lax.axis_index("s")
    w = sid * NC + cid
    pltpu.async_copy(g_hbm.at[pl.ds(sid * RPS, RPS)],
                     g_sh.at[pl.ds(sid * RPS, RPS)], sg1)
    pltpu.async_copy(g_hbm.at[pl.ds(sid * RPS, RPS)],
                     acc_sh.at[pl.ds(sid * RPS, RPS)], sg0)
    pltpu.sync_copy(ei_hbm.at[0, pl.ds(w * EPW, EPW)], src_v)
    pltpu.sync_copy(ei_hbm.at[1, pl.ds(w * EPW, EPW)], dst_v)
    pltpu.make_async_copy(g_hbm.at[pl.ds(sid * RPS, RPS)],
                          g_sh.at[pl.ds(sid * RPS, RPS)], sg1).wait()
    pltpu.make_async_copy(g_hbm.at[pl.ds(sid * RPS, RPS)],
                          acc_sh.at[pl.ds(sid * RPS, RPS)], sg0).wait()
    plsc.subcore_barrier()

    pltpu.async_copy(g_sh.at[src_v.at[pl.ds(0, CHUNK)]], r0, sg0)

    @pl.loop(0, CPW, step=2)
    def _(j):
        pltpu.async_copy(g_sh.at[src_v.at[pl.ds((j + 1) * CHUNK, CHUNK)]],
                         r1, sg1)
        pltpu.make_async_copy(g_sh.at[src_v.at[pl.ds(j * CHUNK, CHUNK)]],
                              r0, sg0).wait()
        pltpu.sync_copy(r0, acc_sh.at[dst_v.at[pl.ds(j * CHUNK, CHUNK)]],
                        add=True)

        @pl.when(j + 2 < CPW)
        def _():
            pltpu.async_copy(
                g_sh.at[src_v.at[pl.ds((j + 2) * CHUNK, CHUNK)]], r0, sg0)

        pltpu.make_async_copy(
            g_sh.at[src_v.at[pl.ds((j + 1) * CHUNK, CHUNK)]], r1, sg1).wait()
        pltpu.sync_copy(r1, acc_sh.at[dst_v.at[pl.ds((j + 1) * CHUNK, CHUNK)]],
                        add=True)

    plsc.subcore_barrier()
    pltpu.sync_copy(acc_sh.at[pl.ds(sid * RPS, RPS)],
                    out_hbm.at[cid, pl.ds(sid * RPS, RPS)])


def _sc_deg(ei32, ones2):
    return pl.kernel(
        _deg_body,
        out_type=jax.ShapeDtypeStruct((NC, N_PAD, H), jnp.float32),
        mesh=_mesh,
        scratch_types=[
            pltpu.VMEM((EPW,), jnp.int32),
            pltpu.VMEM((CHUNK, H), jnp.float32),
            pltpu.VMEM_SHARED((N_PAD, H), jnp.float32),
            pltpu.SemaphoreType.DMA,
        ],
        compiler_params=_sc_params,
    )(ei32, ones2)


def _sc_agg(g, ei32):
    return pl.kernel(
        _agg_body,
        out_type=jax.ShapeDtypeStruct((NC, N_PAD, H), jnp.float32),
        mesh=_mesh,
        scratch_types=[
            pltpu.VMEM((EPW,), jnp.int32),
            pltpu.VMEM((EPW,), jnp.int32),
            pltpu.VMEM((CHUNK, H), jnp.float32),
            pltpu.VMEM((CHUNK, H), jnp.float32),
            pltpu.VMEM_SHARED((N_PAD, H), jnp.float32),
            pltpu.VMEM_SHARED((N_PAD, H), jnp.float32),
            pltpu.SemaphoreType.DMA,
            pltpu.SemaphoreType.DMA,
        ],
        compiler_params=_sc_params,
    )(g, ei32)


def _s1_body(deg_ref, xs_ref, w_ref, g_ref, dis_ref):
    h = jnp.dot(xs_ref[...], w_ref[...], preferred_element_type=jnp.float32)
    deg = deg_ref[0] + deg_ref[1] - 1.0
    dis = lax.rsqrt(deg)
    dis_ref[...] = dis
    g_ref[...] = dis * h


def _tc_mm_scale(deg2v, xs, W1s):
    return pl.pallas_call(
        _s1_body,
        grid=(GRID_TC,),
        in_specs=[pl.BlockSpec((NC, ROWS8, 128), lambda i: (0, i, 0)),
                  pl.BlockSpec((ROWS8, 8 * DF), lambda i: (i, 0)),
                  pl.BlockSpec((8 * DF, 128), lambda i: (0, 0))],
        out_specs=[pl.BlockSpec((ROWS8, 128), lambda i: (i, 0)),
                   pl.BlockSpec((ROWS8, 128), lambda i: (i, 0))],
        out_shape=[jax.ShapeDtypeStruct((NP8, 128), jnp.float32),
                   jax.ShapeDtypeStruct((NP8, 128), jnp.float32)],
    )(deg2v, xs, W1s)


def _s2_body(a_ref, g1_ref, dis_ref, w2_ref, b1_ref, g2_ref):
    dis = dis_ref[...]
    pre = dis * (a_ref[0] + a_ref[1] - g1_ref[...]) + b1_ref[...]
    h = jnp.maximum(pre, 0.0)
    h2 = jnp.dot(h, w2_ref[...], preferred_element_type=jnp.float32)
    g2_ref[...] = dis * h2


def _tc_layer2(agg1v, g1v, dis, W2bd, b1t):
    return pl.pallas_call(
        _s2_body,
        grid=(GRID_TC,),
        in_specs=[pl.BlockSpec((NC, ROWS8, 128), lambda i: (0, i, 0)),
                  pl.BlockSpec((ROWS8, 128), lambda i: (i, 0)),
                  pl.BlockSpec((ROWS8, 128), lambda i: (i, 0)),
                  pl.BlockSpec((128, 128), lambda i: (0, 0)),
                  pl.BlockSpec((1, 128), lambda i: (0, 0))],
        out_specs=pl.BlockSpec((ROWS8, 128), lambda i: (i, 0)),
        out_shape=jax.ShapeDtypeStruct((NP8, 128), jnp.float32),
    )(agg1v, g1v, dis, W2bd, b1t)


def _s3_body(a_ref, g2_ref, dis_ref, b2_ref, seg_ref, o_ref):
    logits = (dis_ref[...] * (a_ref[0] + a_ref[1] - g2_ref[...])
              + b2_ref[...])
    col = lax.broadcasted_iota(jnp.int32, logits.shape, 1)
    masked = jnp.where(col % H < C, logits, -jnp.inf)
    m = jnp.max(masked, axis=1, keepdims=True)
    e = jnp.exp(masked - m)
    s = jnp.dot(e, seg_ref[...], preferred_element_type=jnp.float32)
    o_ref[...] = logits - (m + jnp.log(s))


def _tc_logsoftmax(agg2v, g2v, dis, b2t, seg):
    return pl.pallas_call(
        _s3_body,
        grid=(GRID_TC,),
        in_specs=[pl.BlockSpec((NC, ROWS8, 128), lambda i: (0, i, 0)),
                  pl.BlockSpec((ROWS8, 128), lambda i: (i, 0)),
                  pl.BlockSpec((ROWS8, 128), lambda i: (i, 0)),
                  pl.BlockSpec((1, 128), lambda i: (0, 0)),
                  pl.BlockSpec((128, 128), lambda i: (0, 0))],
        out_specs=pl.BlockSpec((ROWS8, 128), lambda i: (i, 0)),
        out_shape=jax.ShapeDtypeStruct((NP8, 128), jnp.float32),
    )(agg2v, g2v, dis, b2t, seg)


def kernel(x, edge_index, W1, b1, W2, b2):
    ei32 = edge_index.astype(jnp.int32)

    x16 = jnp.pad(x.astype(jnp.bfloat16), ((0, N_PAD - N), (0, 0)))
    xs = x16.reshape(NP8, 8 * DF)
    eye8 = jnp.eye(8, dtype=jnp.float32)
    W1s = jnp.kron(eye8, W1).astype(jnp.bfloat16)
    W2p = jnp.zeros((H, H), jnp.float32).at[:, :C].set(W2)
    W2bd = jnp.kron(eye8, W2p)
    ones2 = jnp.ones((N_PAD, H), jnp.float32)
    b1t = jnp.tile(b1, 8).reshape(1, 128)
    b2t = jnp.tile(jnp.zeros((H,), jnp.float32).at[:C].set(b2), 8)
    b2t = b2t.reshape(1, 128)
    lane = jnp.arange(128)
    seg = (lane[:, None] // H == lane[None, :] // H).astype(jnp.float32)

    deg2 = _sc_deg(ei32, ones2)
    deg2v = deg2.reshape(NC, NP8, 128)
    g1v, dis = _tc_mm_scale(deg2v, xs, W1s)
    agg1 = _sc_agg(g1v.reshape(N_PAD, H), ei32)
    g2v = _tc_layer2(agg1.reshape(NC, NP8, 128), g1v, dis, W2bd, b1t)
    agg2 = _sc_agg(g2v.reshape(N_PAD, H), ei32)
    out128 = _tc_logsoftmax(agg2.reshape(NC, NP8, 128), g2v, dis, b2t, seg)
    return out128.reshape(N_PAD, H)[:N, :C]


# speedup vs baseline: 91.6552x; 1.0099x over previous
import jax
import jax.numpy as jnp
from jax import lax
from jax.experimental import pallas as pl
from jax.experimental.pallas import tpu as pltpu
from jax.experimental.pallas import tpu_sc as plsc

N = 10000
DF = 128
H = 16
C = 7
E = 320000

NC = 2
NS = 16
CHUNK = 1000
CPW = 10
EPW = CHUNK * CPW
N_PAD = 10240
RPS = N_PAD // NS

ROWS = 1024
ROWS8 = ROWS // 8
GRID_TC = N_PAD // ROWS
NP8 = N_PAD * H // 128

_mesh = plsc.VectorSubcoreMesh(core_axis_name="c", subcore_axis_name="s")
_sc_params = pltpu.CompilerParams(use_tc_tiling_on_sc=False)


def _deg_body(ei_hbm, ones_hbm, out_hbm, dst_v, ones_v, acc_sh, sem):
    cid = lax.axis_index("c")
    sid = lax.axis_index("s")
    w = sid * NC + cid
    pltpu.sync_copy(ones_hbm.at[pl.ds(sid * RPS, RPS)],
                    acc_sh.at[pl.ds(sid * RPS, RPS)])
    pltpu.sync_copy(ones_hbm.at[pl.ds(0, CHUNK)], ones_v)
    pltpu.sync_copy(ei_hbm.at[1, pl.ds(w * EPW, EPW)], dst_v)
    plsc.subcore_barrier()

    @pl.loop(0, CPW)
    def _(j):
        pltpu.async_copy(ones_v, acc_sh.at[dst_v.at[pl.ds(j * CHUNK, CHUNK)]],
                         sem, add=True)

    @pl.loop(0, CPW)
    def _(j):
        pltpu.make_async_copy(
            ones_v, acc_sh.at[dst_v.at[pl.ds(j * CHUNK, CHUNK)]], sem).wait()

    plsc.subcore_barrier()
    pltpu.sync_copy(acc_sh.at[pl.ds(sid * RPS, RPS)],
                    out_hbm.at[cid, pl.ds(sid * RPS, RPS)])


def _agg_body(g_hbm, ei_hbm, out_hbm,
              src_v, dst_v, r0, r1, g_sh, acc_sh, sg0, sg1):
    cid = lax.axis_index("c")
    sid = lax.axis_index("s")
    w = sid * NC + cid
    pltpu.async_copy(g_hbm.at[pl.ds(sid * RPS, RPS)],
                     g_sh.at[pl.ds(sid * RPS, RPS)], sg1)
    pltpu.async_copy(g_hbm.at[pl.ds(sid * RPS, RPS)],
                     acc_sh.at[pl.ds(sid * RPS, RPS)], sg0)
    pltpu.sync_copy(ei_hbm.at[0, pl.ds(w * EPW, EPW)], src_v)
    pltpu.sync_copy(ei_hbm.at[1, pl.ds(w * EPW, EPW)], dst_v)
    pltpu.make_async_copy(g_hbm.at[pl.ds(sid * RPS, RPS)],
                          g_sh.at[pl.ds(sid * RPS, RPS)], sg1).wait()
    pltpu.make_async_copy(g_hbm.at[pl.ds(sid * RPS, RPS)],
                          acc_sh.at[pl.ds(sid * RPS, RPS)], sg0).wait()
    plsc.subcore_barrier()

    pltpu.async_copy(g_sh.at[src_v.at[pl.ds(0, CHUNK)]], r0, sg0)

    @pl.loop(0, CPW, step=2)
    def _(j):
        pltpu.async_copy(g_sh.at[src_v.at[pl.ds((j + 1) * CHUNK, CHUNK)]],
                         r1, sg1)
        pltpu.make_async_copy(g_sh.at[src_v.at[pl.ds(j * CHUNK, CHUNK)]],
                              r0, sg0).wait()
        pltpu.sync_copy(r0, acc_sh.at[dst_v.at[pl.ds(j * CHUNK, CHUNK)]],
                        add=True)

        @pl.when(j + 2 < CPW)
        def _():
            pltpu.async_copy(
                g_sh.at[src_v.at[pl.ds((j + 2) * CHUNK, CHUNK)]], r0, sg0)

        pltpu.make_async_copy(
            g_sh.at[src_v.at[pl.ds((j + 1) * CHUNK, CHUNK)]], r1, sg1).wait()
        pltpu.sync_copy(r1, acc_sh.at[dst_v.at[pl.ds((j + 1) * CHUNK, CHUNK)]],
                        add=True)

    plsc.subcore_barrier()
    pltpu.sync_copy(acc_sh.at[pl.ds(sid * RPS, RPS)],
                    out_hbm.at[cid, pl.ds(sid * RPS, RPS)])


def _sc_deg(ei32, ones2):
    return pl.kernel(
        _deg_body,
        out_type=jax.ShapeDtypeStruct((NC, N_PAD, H), jnp.float32),
        mesh=_mesh,
        scratch_types=[
            pltpu.VMEM((EPW,), jnp.int32),
            pltpu.VMEM((CHUNK, H), jnp.float32),
            pltpu.VMEM_SHARED((N_PAD, H), jnp.float32),
            pltpu.SemaphoreType.DMA,
        ],
        compiler_params=_sc_params,
    )(ei32, ones2)


def _sc_agg(g, ei32):
    return pl.kernel(
        _agg_body,
        out_type=jax.ShapeDtypeStruct((NC, N_PAD, H), jnp.float32),
        mesh=_mesh,
        scratch_types=[
            pltpu.VMEM((EPW,), jnp.int32),
            pltpu.VMEM((EPW,), jnp.int32),
            pltpu.VMEM((CHUNK, H), jnp.float32),
            pltpu.VMEM((CHUNK, H), jnp.float32),
            pltpu.VMEM_SHARED((N_PAD, H), jnp.float32),
            pltpu.VMEM_SHARED((N_PAD, H), jnp.float32),
            pltpu.SemaphoreType.DMA,
            pltpu.SemaphoreType.DMA,
        ],
        compiler_params=_sc_params,
    )(g, ei32)


def _mm_body(xs_ref, w_ref, h_ref):
    h_ref[...] = jnp.dot(xs_ref[...], w_ref[...],
                         preferred_element_type=jnp.float32)


def _tc_mm(xs, W1s):
    return pl.pallas_call(
        _mm_body,
        grid=(GRID_TC,),
        in_specs=[pl.BlockSpec((ROWS8, 8 * DF), lambda i: (i, 0)),
                  pl.BlockSpec((8 * DF, 128), lambda i: (0, 0))],
        out_specs=pl.BlockSpec((ROWS8, 128), lambda i: (i, 0)),
        out_shape=jax.ShapeDtypeStruct((NP8, 128), jnp.float32),
    )(xs, W1s)


def _s1_body(deg_ref, h_ref, g_ref, dis_ref):
    deg = deg_ref[0] + deg_ref[1] - 1.0
    dis = lax.rsqrt(deg)
    dis_ref[...] = dis
    g_ref[...] = dis * h_ref[...]


def _tc_scale(deg2v, hv):
    return pl.pallas_call(
        _s1_body,
        grid=(GRID_TC,),
        in_specs=[pl.BlockSpec((NC, ROWS8, 128), lambda i: (0, i, 0)),
                  pl.BlockSpec((ROWS8, 128), lambda i: (i, 0))],
        out_specs=[pl.BlockSpec((ROWS8, 128), lambda i: (i, 0)),
                   pl.BlockSpec((ROWS8, 128), lambda i: (i, 0))],
        out_shape=[jax.ShapeDtypeStruct((NP8, 128), jnp.float32),
                   jax.ShapeDtypeStruct((NP8, 128), jnp.float32)],
    )(deg2v, hv)


def _s2_body(a_ref, g1_ref, dis_ref, w2_ref, b1_ref, g2_ref):
    dis = dis_ref[...]
    pre = dis * (a_ref[0] + a_ref[1] - g1_ref[...]) + b1_ref[...]
    h = jnp.maximum(pre, 0.0)
    h2 = jnp.dot(h, w2_ref[...], preferred_element_type=jnp.float32)
    g2_ref[...] = dis * h2


def _tc_layer2(agg1v, g1v, dis, W2bd, b1t):
    return pl.pallas_call(
        _s2_body,
        grid=(GRID_TC,),
        in_specs=[pl.BlockSpec((NC, ROWS8, 128), lambda i: (0, i, 0)),
                  pl.BlockSpec((ROWS8, 128), lambda i: (i, 0)),
                  pl.BlockSpec((ROWS8, 128), lambda i: (i, 0)),
                  pl.BlockSpec((128, 128), lambda i: (0, 0)),
                  pl.BlockSpec((1, 128), lambda i: (0, 0))],
        out_specs=pl.BlockSpec((ROWS8, 128), lambda i: (i, 0)),
        out_shape=jax.ShapeDtypeStruct((NP8, 128), jnp.float32),
    )(agg1v, g1v, dis, W2bd, b1t)


def _s3_body(a_ref, g2_ref, dis_ref, b2_ref, seg_ref, o_ref):
    logits = (dis_ref[...] * (a_ref[0] + a_ref[1] - g2_ref[...])
              + b2_ref[...])
    col = lax.broadcasted_iota(jnp.int32, logits.shape, 1)
    masked = jnp.where(col % H < C, logits, -jnp.inf)
    m = jnp.max(masked, axis=1, keepdims=True)
    e = jnp.exp(masked - m)
    s = jnp.dot(e, seg_ref[...], preferred_element_type=jnp.float32)
    o_ref[...] = logits - (m + jnp.log(s))


def _tc_logsoftmax(agg2v, g2v, dis, b2t, seg):
    return pl.pallas_call(
        _s3_body,
        grid=(GRID_TC,),
        in_specs=[pl.BlockSpec((NC, ROWS8, 128), lambda i: (0, i, 0)),
                  pl.BlockSpec((ROWS8, 128), lambda i: (i, 0)),
                  pl.BlockSpec((ROWS8, 128), lambda i: (i, 0)),
                  pl.BlockSpec((1, 128), lambda i: (0, 0)),
                  pl.BlockSpec((128, 128), lambda i: (0, 0))],
        out_specs=pl.BlockSpec((ROWS8, 128), lambda i: (i, 0)),
        out_shape=jax.ShapeDtypeStruct((NP8, 128), jnp.float32),
    )(agg2v, g2v, dis, b2t, seg)


def kernel(x, edge_index, W1, b1, W2, b2):
    ei32 = edge_index.astype(jnp.int32)

    x16 = jnp.pad(x.astype(jnp.bfloat16), ((0, N_PAD - N), (0, 0)))
    xs = x16.reshape(NP8, 8 * DF)
    eye8 = jnp.eye(8, dtype=jnp.float32)
    W1s = jnp.kron(eye8, W1).astype(jnp.bfloat16)
    W2p = jnp.zeros((H, H), jnp.float32).at[:, :C].set(W2)
    W2bd = jnp.kron(eye8, W2p)
    ones2 = jnp.ones((N_PAD, H), jnp.float32)
    b1t = jnp.tile(b1, 8).reshape(1, 128)
    b2t = jnp.tile(jnp.zeros((H,), jnp.float32).at[:C].set(b2), 8)
    b2t = b2t.reshape(1, 128)
    lane = jnp.arange(128)
    seg = (lane[:, None] // H == lane[None, :] // H).astype(jnp.float32)

    deg2 = _sc_deg(ei32, ones2)
    hv = _tc_mm(xs, W1s)
    deg2v = deg2.reshape(NC, NP8, 128)
    g1v, dis = _tc_scale(deg2v, hv)
    agg1 = _sc_agg(g1v.reshape(N_PAD, H), ei32)
    g2v = _tc_layer2(agg1.reshape(NC, NP8, 128), g1v, dis, W2bd, b1t)
    agg2 = _sc_agg(g2v.reshape(N_PAD, H), ei32)
    out128 = _tc_logsoftmax(agg2.reshape(NC, NP8, 128), g2v, dis, b2t, seg)
    return out128.reshape(N_PAD, H)[:N, :C]
